# Initial kernel scaffold; baseline (speedup 1.0000x reference)
import jax
import jax.numpy as jnp
from jax.experimental import pallas as pl


def kernel(player_ids, ball_feat, query_feat, src_pb, dst_pb, src_bq, dst_bq, src_pq, dst_pq, player_table, W_p, b_p, W_b, b_b, W_q, b_q, W_msg_pb, W_msg_bq, W_msg_pq, ln_ball_g, ln_ball_b, ln_q_g, ln_q_b, P1_w, P1_b, ln_p1_g, ln_p1_b, P2_w, P2_b, ln_p2_g, ln_p2_b, P3_w, P3_b):
    raise NotImplementedError("write your pallas kernel here")


# speedup vs baseline: 7.7075x; 7.7075x over previous
import functools

import jax
import jax.numpy as jnp
from jax import lax
from jax.experimental import pallas as pl
from jax.experimental.pallas import tpu as pltpu
from jax.experimental.pallas import tpu_sc as plsc

H = 64
L = 3
NB = 50000
NQ = 1024
NTAB = 5008
NPID = 51200

NC = 2
NS = 16
NW = NC * NS

C = 512

E_PB = 800000
E_PB_PAD = 819200
PB_PER_W = E_PB_PAD // NW
PB_NCHUNK = PB_PER_W // C

E_BQ = 262144
BQ_PER_W = E_BQ // NW
BQ_CHUNK = 1024
BQ_NCHUNK = BQ_PER_W // BQ_CHUNK
BQD_NCHUNK = BQ_PER_W // C

E_PQ = 22528
E_PQ_PAD = 32768
PQ_PER_W = E_PQ_PAD // NW
PQ_NCHUNK = PQ_PER_W // C

GROW_PER_W = NPID // NW
GCHUNK = 400

ACC_ROWS = 50048
ZROWS = ACC_ROWS // NS
QROWS = NQ // NS

f32 = jnp.float32
i32 = jnp.int32


def _ln(x, g, b):
    mu = jnp.mean(x, axis=-1, keepdims=True)
    xc = x - mu
    var = jnp.mean(xc * xc, axis=-1, keepdims=True)
    return xc * jax.lax.rsqrt(var + 1e-5) * g + b


def _k1_body(pt_ref, wp_ref, bp_ref, e0_ref, e1_ref):
    x = jax.nn.gelu(
        jnp.dot(pt_ref[...], wp_ref[...], preferred_element_type=f32)
        + bp_ref[...])
    e0_ref[...] = x[:, :32]
    e1_ref[...] = x[:, 32:]


def _k1(pt_pad, W_p, b_p):
    return pl.pallas_call(
        _k1_body,
        out_shape=[jax.ShapeDtypeStruct((NTAB, 32), f32),
                   jax.ShapeDtypeStruct((NTAB, 32), f32)],
    )(pt_pad, W_p, b_p)


_SC_PARAMS = pltpu.CompilerParams(needs_layout_passes=False,
                                  use_tc_tiling_on_sc=False)


def _mesh():
    return plsc.VectorSubcoreMesh(core_axis_name="c", subcore_axis_name="s")


def _k1b_body(enc0, enc1, pid, encf0, encf1, pid_v, rows_v, sem):
    c = lax.axis_index("c")
    s = lax.axis_index("s")
    w = s * NC + c
    for tab, outf in ((enc0, encf0), (enc1, encf1)):
        for j in range(GROW_PER_W // GCHUNK):
            base = w * GROW_PER_W + j * GCHUNK
            pltpu.sync_copy(pid.at[pl.ds(base, GCHUNK)], pid_v)
            pltpu.async_copy(tab.at[pid_v],
                             rows_v.at[pl.ds(0, GCHUNK), :], sem).wait()
            pltpu.sync_copy(rows_v.at[pl.ds(0, GCHUNK), :],
                            outf.at[pl.ds(base, GCHUNK), :])


def _k1b(enc0, enc1, pid):
    return pl.kernel(
        _k1b_body,
        out_type=[jax.ShapeDtypeStruct((NPID, 32), f32),
                  jax.ShapeDtypeStruct((NPID, 32), f32)],
        mesh=_mesh(),
        compiler_params=_SC_PARAMS,
        scratch_types=[
            pltpu.VMEM((GCHUNK,), i32),
            pltpu.VMEM((GCHUNK, 32), f32),
            pltpu.SemaphoreType.DMA,
        ],
    )(enc0, enc1, pid)


def _k2_body(encf0, encf1, srcpb, dstpb, dstbq, srcpq, dstpq,
             zeros50, ones512,
             spb, degb, sq, degq2, degq1,
             acc, src_v, dst_v, rows_v, sem):
    c = lax.axis_index("c")
    s = lax.axis_index("s")
    w = s * NC + c

    def barrier():
        plsc.subcore_barrier()

    def phase(table, src_arr, dst_arr, per_w, nchunk, zrows, dump_to):
        pltpu.sync_copy(zeros50.at[pl.ds(0, zrows), :],
                        acc.at[pl.ds(s * zrows, zrows), :])
        barrier()

        def chunk(j, carry):
            base = w * per_w + j * C
            pltpu.sync_copy(dst_arr.at[pl.ds(base, C)], dst_v)
            if table is not None:
                pltpu.sync_copy(src_arr.at[pl.ds(base, C)], src_v)
                pltpu.async_copy(table.at[src_v], rows_v, sem).wait()
            pltpu.sync_copy(rows_v, acc.at[dst_v], add=True)
            return carry

        lax.fori_loop(0, nchunk, chunk, 0)
        barrier()
        pltpu.sync_copy(acc.at[pl.ds(s * zrows, zrows), :], dump_to)
        barrier()

    def fill_ones():
        pltpu.sync_copy(ones512, rows_v)

    phase(encf0, srcpb, dstpb, PB_PER_W, PB_NCHUNK, ZROWS,
          spb.at[c, 0, pl.ds(s * ZROWS, ZROWS), :])
    phase(encf1, srcpb, dstpb, PB_PER_W, PB_NCHUNK, ZROWS,
          spb.at[c, 1, pl.ds(s * ZROWS, ZROWS), :])
    fill_ones()
    phase(None, None, dstpb, PB_PER_W, PB_NCHUNK, ZROWS,
          degb.at[c, pl.ds(s * ZROWS, ZROWS), :])

    phase(None, None, dstbq, BQ_PER_W, BQD_NCHUNK, QROWS,
          degq1.at[c, pl.ds(s * QROWS, QROWS), :])

    phase(encf0, srcpq, dstpq, PQ_PER_W, PQ_NCHUNK, QROWS,
          sq.at[c, 0, pl.ds(s * QROWS, QROWS), :])
    phase(encf1, srcpq, dstpq, PQ_PER_W, PQ_NCHUNK, QROWS,
          sq.at[c, 1, pl.ds(s * QROWS, QROWS), :])
    fill_ones()
    phase(None, None, dstpq, PQ_PER_W, PQ_NCHUNK, QROWS,
          degq2.at[c, pl.ds(s * QROWS, QROWS), :])


def _k2(encf0, encf1, srcpb, dstpb, dstbq, srcpq, dstpq, zeros50, ones512):
    return pl.kernel(
        _k2_body,
        out_type=[
            jax.ShapeDtypeStruct((NC, 2, ACC_ROWS, 32), f32),
            jax.ShapeDtypeStruct((NC, ACC_ROWS, 32), f32),
            jax.ShapeDtypeStruct((NC, 2, NQ, 32), f32),
            jax.ShapeDtypeStruct((NC, NQ, 32), f32),
            jax.ShapeDtypeStruct((NC, NQ, 32), f32),
        ],
        mesh=_mesh(),
        compiler_params=_SC_PARAMS,
        scratch_types=[
            pltpu.VMEM_SHARED((ACC_ROWS, 32), f32),
            pltpu.VMEM((C,), i32),
            pltpu.VMEM((C,), i32),
            pltpu.VMEM((C, 32), f32),
            pltpu.SemaphoreType.DMA,
        ],
    )(encf0, encf1, srcpb, dstpb, dstbq, srcpq, dstpq, zeros50, ones512)


K3_R = 2000


def _k3_body(bf_ref, spb_ref, degb_ref, wb_ref, bb_ref, wmsg_ref,
             lng_ref, lnb_ref, o1_ref, o2_ref, o3_ref):
    x = jax.nn.gelu(
        jnp.dot(bf_ref[...], wb_ref[...], preferred_element_type=f32)
        + bb_ref[...])
    deg = jnp.maximum(degb_ref[0, :, 0] + degb_ref[1, :, 0], 1.0)[:, None]
    s0 = spb_ref[0, 0] + spb_ref[1, 0]
    s1 = spb_ref[0, 1] + spb_ref[1, 1]
    S = jnp.concatenate([s0, s1], axis=1) / deg
    outs = (o1_ref, o2_ref, o3_ref)
    for l in range(L):
        m = jnp.dot(S, wmsg_ref[l], preferred_element_type=f32)
        x = _ln(x + jax.nn.gelu(m), lng_ref[l], lnb_ref[l])
        outs[l][...] = x


def _k3(ball_feat, spb, degb, W_b, b_b, W_msg_pb, ln_ball_g, ln_ball_b):
    grid = (NB // K3_R,)
    return pl.pallas_call(
        _k3_body,
        grid=grid,
        in_specs=[
            pl.BlockSpec((K3_R, 16), lambda i: (i, 0)),
            pl.BlockSpec((NC, 2, K3_R, 32), lambda i: (0, 0, i, 0)),
            pl.BlockSpec((NC, K3_R, 32), lambda i: (0, i, 0)),
            pl.BlockSpec((16, H), lambda i: (0, 0)),
            pl.BlockSpec((1, H), lambda i: (0, 0)),
            pl.BlockSpec((L, H, H), lambda i: (0, 0, 0)),
            pl.BlockSpec((L, H), lambda i: (0, 0)),
            pl.BlockSpec((L, H), lambda i: (0, 0)),
        ],
        out_specs=[pl.BlockSpec((K3_R, H), lambda i: (i, 0))] * 3,
        out_shape=[jax.ShapeDtypeStruct((NB, H), f32)] * 3,
    )(ball_feat, spb, degb, W_b, b_b, W_msg_pb, ln_ball_g, ln_ball_b)


def _k4_body(hb1, hb2, hb3, srcbq, dstbq, zerosq,
             tout, accq, idx1024, dst1024, rows_v, sem):
    c = lax.axis_index("c")
    s = lax.axis_index("s")
    w = s * NC + c

    for l, hb in enumerate((hb1, hb2, hb3)):
        pltpu.sync_copy(zerosq, accq.at[pl.ds(s * QROWS, QROWS), :])
        plsc.subcore_barrier()

        def chunk(j, carry, hb=hb):
            base = w * BQ_PER_W + j * BQ_CHUNK
            pltpu.sync_copy(srcbq.at[pl.ds(base, BQ_CHUNK)], idx1024)
            pltpu.sync_copy(dstbq.at[pl.ds(base, BQ_CHUNK)], dst1024)
            pltpu.async_copy(hb.at[idx1024], rows_v, sem).wait()
            pltpu.sync_copy(rows_v, accq.at[dst1024], add=True)
            return carry

        lax.fori_loop(0, BQ_NCHUNK, chunk, 0)
        plsc.subcore_barrier()
        pltpu.sync_copy(accq.at[pl.ds(s * QROWS, QROWS), :],
                        tout.at[l, c, pl.ds(s * QROWS, QROWS), :])
        plsc.subcore_barrier()


def _k4(hb1, hb2, hb3, srcbq, dstbq, zerosq):
    return pl.kernel(
        _k4_body,
        out_type=jax.ShapeDtypeStruct((L, NC, NQ, H), f32),
        mesh=_mesh(),
        compiler_params=_SC_PARAMS,
        scratch_types=[
            pltpu.VMEM_SHARED((NQ, H), f32),
            pltpu.VMEM((BQ_CHUNK,), i32),
            pltpu.VMEM((BQ_CHUNK,), i32),
            pltpu.VMEM((BQ_CHUNK, H), f32),
            pltpu.SemaphoreType.DMA,
        ],
    )(hb1, hb2, hb3, srcbq, dstbq, zerosq)


def _k5_body(qf_ref, t_ref, sq_ref, dq1_ref, dq2_ref, wq_ref, bq_ref,
             wbq_ref, wpq_ref, lng_ref, lnb_ref,
             p1w_ref, p1b_ref, g1_ref, b1_ref,
             p2w_ref, p2b_ref, g2_ref, b2_ref,
             p3w_ref, p3b_ref, out_ref):
    hq = jax.nn.gelu(
        jnp.dot(qf_ref[...], wq_ref[...], preferred_element_type=f32)
        + bq_ref[...])
    dq1 = jnp.maximum(dq1_ref[0, :, 0] + dq1_ref[1, :, 0], 1.0)[:, None]
    dq2 = jnp.maximum(dq2_ref[0, :, 0] + dq2_ref[1, :, 0], 1.0)[:, None]
    s0 = sq_ref[0, 0] + sq_ref[1, 0]
    s1 = sq_ref[0, 1] + sq_ref[1, 1]
    Spq = jnp.concatenate([s0, s1], axis=1) / dq2
    for l in range(L):
        Tl = (t_ref[l, 0] + t_ref[l, 1]) / dq1
        a = (jnp.dot(Tl, wbq_ref[l], preferred_element_type=f32)
             + jnp.dot(Spq, wpq_ref[l], preferred_element_type=f32))
        hq = _ln(hq + jax.nn.gelu(a), lng_ref[l], lnb_ref[l])
    z = jax.nn.gelu(_ln(
        jnp.dot(hq, p1w_ref[...], preferred_element_type=f32) + p1b_ref[...],
        g1_ref[...], b1_ref[...]))
    z = jax.nn.gelu(_ln(
        jnp.dot(z, p2w_ref[...], preferred_element_type=f32) + p2b_ref[...],
        g2_ref[...], b2_ref[...]))
    out_ref[...] = (jnp.dot(z, p3w_ref[...], preferred_element_type=f32)
                    + p3b_ref[...])


def _k5(query_feat, T, sq, degq1, degq2, W_q, b_q, W_msg_bq, W_msg_pq,
        ln_q_g, ln_q_b, P1_w, P1_b, ln_p1_g, ln_p1_b,
        P2_w, P2_b, ln_p2_g, ln_p2_b, P3_w8, P3_b8):
    return pl.pallas_call(
        _k5_body,
        out_shape=jax.ShapeDtypeStruct((NQ, 8), f32),
    )(query_feat, T, sq, degq1, degq2, W_q, b_q, W_msg_bq, W_msg_pq,
      ln_q_g, ln_q_b, P1_w, P1_b, ln_p1_g, ln_p1_b,
      P2_w, P2_b, ln_p2_g, ln_p2_b, P3_w8, P3_b8)


def kernel(player_ids, ball_feat, query_feat, src_pb, dst_pb, src_bq, dst_bq,
           src_pq, dst_pq, player_table, W_p, b_p, W_b, b_b, W_q, b_q,
           W_msg_pb, W_msg_bq, W_msg_pq, ln_ball_g, ln_ball_b, ln_q_g, ln_q_b,
           P1_w, P1_b, ln_p1_g, ln_p1_b, P2_w, P2_b, ln_p2_g, ln_p2_b,
           P3_w, P3_b):
    player_ids = player_ids.astype(i32)
    src_pb = src_pb.astype(i32)
    dst_pb = dst_pb.astype(i32)
    src_bq = src_bq.astype(i32)
    dst_bq = dst_bq.astype(i32)
    src_pq = src_pq.astype(i32)
    dst_pq = dst_pq.astype(i32)

    pt_pad = jnp.pad(player_table, ((0, NTAB - 5000), (0, 0)))
    enc0, enc1 = _k1(pt_pad, W_p, b_p.reshape(1, H))

    npad = E_PB_PAD - E_PB
    npad_q = E_PQ_PAD - E_PQ
    pid_p = jnp.concatenate([player_ids, jnp.full((NPID - 50000,), 0, i32)])
    srcpb_p = jnp.concatenate([src_pb, jnp.full((npad,), 0, i32)])
    dstpb_p = jnp.concatenate([dst_pb, jnp.full((npad,), NB, i32)])
    srcpq_p = jnp.concatenate([src_pq, jnp.full((npad_q,), 0, i32)])
    dstpq_p = jnp.concatenate([dst_pq, jnp.full((npad_q,), NQ, i32)])

    zeros50 = jnp.zeros((ZROWS, 32), f32)
    ones512 = jnp.ones((C, 32), f32)
    zerosq = jnp.zeros((QROWS, H), f32)

    encf0, encf1 = _k1b(enc0, enc1, pid_p)
    spb, degb, sq, degq2, degq1 = _k2(
        encf0, encf1, srcpb_p, dstpb_p, dst_bq, srcpq_p, dstpq_p,
        zeros50, ones512)

    hb1, hb2, hb3 = _k3(ball_feat, spb, degb, W_b, b_b.reshape(1, H),
                        W_msg_pb, ln_ball_g, ln_ball_b)

    T = _k4(hb1, hb2, hb3, src_bq, dst_bq, zerosq)

    P3_w8 = jnp.pad(P3_w, ((0, 0), (0, 1)))
    P3_b8 = jnp.pad(P3_b, ((0, 1))).reshape(1, 8)
    out8 = _k5(query_feat, T, sq, degq1, degq2,
               W_q, b_q.reshape(1, H), W_msg_bq, W_msg_pq, ln_q_g, ln_q_b,
               P1_w, P1_b.reshape(1, H), ln_p1_g.reshape(1, H),
               ln_p1_b.reshape(1, H), P2_w, P2_b.reshape(1, 32),
               ln_p2_g.reshape(1, 32), ln_p2_b.reshape(1, 32), P3_w8, P3_b8)
    return out8[:, :7]


# speedup vs baseline: 7.7889x; 1.0106x over previous
import functools

import jax
import jax.numpy as jnp
from jax import lax
from jax.experimental import pallas as pl
from jax.experimental.pallas import tpu as pltpu
from jax.experimental.pallas import tpu_sc as plsc

H = 64
L = 3
NB = 50000
NQ = 1024
NTAB = 5008
NPID = 51200

NC = 2
NS = 16
NW = NC * NS

C = 256

E_PB = 800000
E_PB_PAD = 819200
PB_PER_W = E_PB_PAD // NW
PB_NCHUNK = PB_PER_W // C

E_BQ = 262144
BQ_PER_W = E_BQ // NW
BQ_NCHUNK = BQ_PER_W // C

E_PQ = 22528
E_PQ_PAD = 32768
PQ_PER_W = E_PQ_PAD // NW
PQ_NCHUNK = PQ_PER_W // C

GROW_PER_W = NPID // NW
GCHUNK = 400

ACC_ROWS = 50048
ZROWS = ACC_ROWS // NS
QROWS = NQ // NS

f32 = jnp.float32
i32 = jnp.int32


def _ln(x, g, b):
    mu = jnp.mean(x, axis=-1, keepdims=True)
    xc = x - mu
    var = jnp.mean(xc * xc, axis=-1, keepdims=True)
    return xc * jax.lax.rsqrt(var + 1e-5) * g + b


def _k1_body(pt_ref, wp_ref, bp_ref, e0_ref, e1_ref):
    x = jax.nn.gelu(
        jnp.dot(pt_ref[...], wp_ref[...], preferred_element_type=f32)
        + bp_ref[...])
    e0_ref[...] = x[:, :32]
    e1_ref[...] = x[:, 32:]


def _k1(pt_pad, W_p, b_p):
    return pl.pallas_call(
        _k1_body,
        out_shape=[jax.ShapeDtypeStruct((NTAB, 32), f32),
                   jax.ShapeDtypeStruct((NTAB, 32), f32)],
    )(pt_pad, W_p, b_p)


_SC_PARAMS = pltpu.CompilerParams(needs_layout_passes=False,
                                  use_tc_tiling_on_sc=False)


def _mesh():
    return plsc.VectorSubcoreMesh(core_axis_name="c", subcore_axis_name="s")


def _k1b_body(enc0, enc1, pid, encf0, encf1, pid_v, rows_v, sem):
    c = lax.axis_index("c")
    s = lax.axis_index("s")
    w = s * NC + c
    for tab, outf in ((enc0, encf0), (enc1, encf1)):
        for j in range(GROW_PER_W // GCHUNK):
            base = w * GROW_PER_W + j * GCHUNK
            pltpu.sync_copy(pid.at[pl.ds(base, GCHUNK)], pid_v)
            pltpu.async_copy(tab.at[pid_v],
                             rows_v.at[pl.ds(0, GCHUNK), :], sem).wait()
            pltpu.sync_copy(rows_v.at[pl.ds(0, GCHUNK), :],
                            outf.at[pl.ds(base, GCHUNK), :])


def _k1b(enc0, enc1, pid):
    return pl.kernel(
        _k1b_body,
        out_type=[jax.ShapeDtypeStruct((NPID, 32), f32),
                  jax.ShapeDtypeStruct((NPID, 32), f32)],
        mesh=_mesh(),
        compiler_params=_SC_PARAMS,
        scratch_types=[
            pltpu.VMEM((GCHUNK,), i32),
            pltpu.VMEM((GCHUNK, 32), f32),
            pltpu.SemaphoreType.DMA,
        ],
    )(enc0, enc1, pid)


def _seg_phase(s, w, edges, table, acc, ib, rows, si, sg, nch,
               zeros_src, zrows, dump_to):
    pltpu.sync_copy(zeros_src, acc.at[pl.ds(s * zrows, zrows), :])
    plsc.subcore_barrier()

    cb = w * nch
    pltpu.async_copy(edges.at[cb], ib[0], si[0])
    pltpu.async_copy(edges.at[cb + 1], ib[1], si[1])
    if table is not None:
        pltpu.make_async_copy(edges.at[cb], ib[0], si[0]).wait()
        pltpu.async_copy(table.at[ib[0].at[0]], rows[0], sg[0])

    def pair(jj, carry):
        for b in (0, 1):
            j = 2 * jj + b
            if table is not None:
                pltpu.make_async_copy(table.at[ib[b].at[0]],
                                      rows[b], sg[b]).wait()

                @pl.when(j + 1 < nch)
                def _():
                    nb = 1 - b
                    pltpu.make_async_copy(edges.at[cb + j + 1],
                                          ib[nb], si[nb]).wait()
                    pltpu.async_copy(table.at[ib[nb].at[0]], rows[nb], sg[nb])

                pltpu.sync_copy(rows[b], acc.at[ib[b].at[1]], add=True)
            else:
                pltpu.make_async_copy(edges.at[cb + j], ib[b], si[b]).wait()
                pltpu.sync_copy(rows[0], acc.at[ib[b].at[1]], add=True)

            @pl.when(j + 2 < nch)
            def _():
                pltpu.async_copy(edges.at[cb + j + 2], ib[b], si[b])
        return carry

    lax.fori_loop(0, nch // 2, pair, 0)
    plsc.subcore_barrier()
    pltpu.sync_copy(acc.at[pl.ds(s * zrows, zrows), :], dump_to)
    plsc.subcore_barrier()


def _k2_body(encf0, encf1, pbe, bqe, pqe, zeros50, ones256,
             spb, degb, sq, degq2, degq1,
             acc, ib0, ib1, rows0, rows1, si0, si1, sg0, sg1):
    c = lax.axis_index("c")
    s = lax.axis_index("s")
    w = s * NC + c
    ib = (ib0, ib1)
    rows = (rows0, rows1)
    si = (si0, si1)
    sg = (sg0, sg1)

    def phase(edges, table, nch, zrows, dump_to):
        _seg_phase(s, w, edges, table, acc, ib, rows, si, sg, nch,
                   zeros50.at[pl.ds(0, zrows), :], zrows, dump_to)

    phase(pbe, encf0, PB_NCHUNK, ZROWS,
          spb.at[c, 0, pl.ds(s * ZROWS, ZROWS), :])
    phase(pbe, encf1, PB_NCHUNK, ZROWS,
          spb.at[c, 1, pl.ds(s * ZROWS, ZROWS), :])
    pltpu.sync_copy(ones256, rows0)
    phase(pbe, None, PB_NCHUNK, ZROWS,
          degb.at[c, pl.ds(s * ZROWS, ZROWS), :])

    phase(bqe, None, BQ_NCHUNK, QROWS,
          degq1.at[c, pl.ds(s * QROWS, QROWS), :])

    phase(pqe, encf0, PQ_NCHUNK, QROWS,
          sq.at[c, 0, pl.ds(s * QROWS, QROWS), :])
    phase(pqe, encf1, PQ_NCHUNK, QROWS,
          sq.at[c, 1, pl.ds(s * QROWS, QROWS), :])
    pltpu.sync_copy(ones256, rows0)
    phase(pqe, None, PQ_NCHUNK, QROWS,
          degq2.at[c, pl.ds(s * QROWS, QROWS), :])


def _k2(encf0, encf1, pbe, bqe, pqe, zeros50, ones256):
    return pl.kernel(
        _k2_body,
        out_type=[
            jax.ShapeDtypeStruct((NC, 2, ACC_ROWS, 32), f32),
            jax.ShapeDtypeStruct((NC, ACC_ROWS, 32), f32),
            jax.ShapeDtypeStruct((NC, 2, NQ, 32), f32),
            jax.ShapeDtypeStruct((NC, NQ, 32), f32),
            jax.ShapeDtypeStruct((NC, NQ, 32), f32),
        ],
        mesh=_mesh(),
        compiler_params=_SC_PARAMS,
        scratch_types=[
            pltpu.VMEM_SHARED((ACC_ROWS, 32), f32),
            pltpu.VMEM((2, C), i32),
            pltpu.VMEM((2, C), i32),
            pltpu.VMEM((C, 32), f32),
            pltpu.VMEM((C, 32), f32),
            pltpu.SemaphoreType.DMA,
            pltpu.SemaphoreType.DMA,
            pltpu.SemaphoreType.DMA,
            pltpu.SemaphoreType.DMA,
        ],
    )(encf0, encf1, pbe, bqe, pqe, zeros50, ones256)


K3_R = 2000


def _k3_body(bf_ref, spb_ref, degb_ref, wb_ref, bb_ref, wmsg_ref,
             lng_ref, lnb_ref, o1_ref, o2_ref, o3_ref):
    x = jax.nn.gelu(
        jnp.dot(bf_ref[...], wb_ref[...], preferred_element_type=f32)
        + bb_ref[...])
    deg = jnp.maximum(degb_ref[0, :, 0] + degb_ref[1, :, 0], 1.0)[:, None]
    s0 = spb_ref[0, 0] + spb_ref[1, 0]
    s1 = spb_ref[0, 1] + spb_ref[1, 1]
    S = jnp.concatenate([s0, s1], axis=1) / deg
    outs = (o1_ref, o2_ref, o3_ref)
    for l in range(L):
        m = jnp.dot(S, wmsg_ref[l], preferred_element_type=f32)
        x = _ln(x + jax.nn.gelu(m), lng_ref[l], lnb_ref[l])
        outs[l][...] = x


def _k3(ball_feat, spb, degb, W_b, b_b, W_msg_pb, ln_ball_g, ln_ball_b):
    grid = (NB // K3_R,)
    return pl.pallas_call(
        _k3_body,
        grid=grid,
        in_specs=[
            pl.BlockSpec((K3_R, 16), lambda i: (i, 0)),
            pl.BlockSpec((NC, 2, K3_R, 32), lambda i: (0, 0, i, 0)),
            pl.BlockSpec((NC, K3_R, 32), lambda i: (0, i, 0)),
            pl.BlockSpec((16, H), lambda i: (0, 0)),
            pl.BlockSpec((1, H), lambda i: (0, 0)),
            pl.BlockSpec((L, H, H), lambda i: (0, 0, 0)),
            pl.BlockSpec((L, H), lambda i: (0, 0)),
            pl.BlockSpec((L, H), lambda i: (0, 0)),
        ],
        out_specs=[pl.BlockSpec((K3_R, H), lambda i: (i, 0))] * 3,
        out_shape=[jax.ShapeDtypeStruct((NB, H), f32)] * 3,
    )(ball_feat, spb, degb, W_b, b_b, W_msg_pb, ln_ball_g, ln_ball_b)


def _k4_body(hb1, hb2, hb3, bqe, zerosq,
             tout, accq, ib0, ib1, rows0, rows1, si0, si1, sg0, sg1):
    c = lax.axis_index("c")
    s = lax.axis_index("s")
    w = s * NC + c
    ib = (ib0, ib1)
    rows = (rows0, rows1)
    si = (si0, si1)
    sg = (sg0, sg1)

    for l, hb in enumerate((hb1, hb2, hb3)):
        _seg_phase(s, w, bqe, hb, accq, ib, rows, si, sg, BQ_NCHUNK,
                   zerosq, QROWS,
                   tout.at[l, c, pl.ds(s * QROWS, QROWS), :])


def _k4(hb1, hb2, hb3, bqe, zerosq):
    return pl.kernel(
        _k4_body,
        out_type=jax.ShapeDtypeStruct((L, NC, NQ, H), f32),
        mesh=_mesh(),
        compiler_params=_SC_PARAMS,
        scratch_types=[
            pltpu.VMEM_SHARED((NQ, H), f32),
            pltpu.VMEM((2, C), i32),
            pltpu.VMEM((2, C), i32),
            pltpu.VMEM((C, H), f32),
            pltpu.VMEM((C, H), f32),
            pltpu.SemaphoreType.DMA,
            pltpu.SemaphoreType.DMA,
            pltpu.SemaphoreType.DMA,
            pltpu.SemaphoreType.DMA,
        ],
    )(hb1, hb2, hb3, bqe, zerosq)


def _k5_body(qf_ref, t_ref, sq_ref, dq1_ref, dq2_ref, wq_ref, bq_ref,
             wbq_ref, wpq_ref, lng_ref, lnb_ref,
             p1w_ref, p1b_ref, g1_ref, b1_ref,
             p2w_ref, p2b_ref, g2_ref, b2_ref,
             p3w_ref, p3b_ref, out_ref):
    hq = jax.nn.gelu(
        jnp.dot(qf_ref[...], wq_ref[...], preferred_element_type=f32)
        + bq_ref[...])
    dq1 = jnp.maximum(dq1_ref[0, :, 0] + dq1_ref[1, :, 0], 1.0)[:, None]
    dq2 = jnp.maximum(dq2_ref[0, :, 0] + dq2_ref[1, :, 0], 1.0)[:, None]
    s0 = sq_ref[0, 0] + sq_ref[1, 0]
    s1 = sq_ref[0, 1] + sq_ref[1, 1]
    Spq = jnp.concatenate([s0, s1], axis=1) / dq2
    for l in range(L):
        Tl = (t_ref[l, 0] + t_ref[l, 1]) / dq1
        a = (jnp.dot(Tl, wbq_ref[l], preferred_element_type=f32)
             + jnp.dot(Spq, wpq_ref[l], preferred_element_type=f32))
        hq = _ln(hq + jax.nn.gelu(a), lng_ref[l], lnb_ref[l])
    z = jax.nn.gelu(_ln(
        jnp.dot(hq, p1w_ref[...], preferred_element_type=f32) + p1b_ref[...],
        g1_ref[...], b1_ref[...]))
    z = jax.nn.gelu(_ln(
        jnp.dot(z, p2w_ref[...], preferred_element_type=f32) + p2b_ref[...],
        g2_ref[...], b2_ref[...]))
    out_ref[...] = (jnp.dot(z, p3w_ref[...], preferred_element_type=f32)
                    + p3b_ref[...])


def _k5(query_feat, T, sq, degq1, degq2, W_q, b_q, W_msg_bq, W_msg_pq,
        ln_q_g, ln_q_b, P1_w, P1_b, ln_p1_g, ln_p1_b,
        P2_w, P2_b, ln_p2_g, ln_p2_b, P3_w8, P3_b8):
    return pl.pallas_call(
        _k5_body,
        out_shape=jax.ShapeDtypeStruct((NQ, 8), f32),
    )(query_feat, T, sq, degq1, degq2, W_q, b_q, W_msg_bq, W_msg_pq,
      ln_q_g, ln_q_b, P1_w, P1_b, ln_p1_g, ln_p1_b,
      P2_w, P2_b, ln_p2_g, ln_p2_b, P3_w8, P3_b8)


def kernel(player_ids, ball_feat, query_feat, src_pb, dst_pb, src_bq, dst_bq,
           src_pq, dst_pq, player_table, W_p, b_p, W_b, b_b, W_q, b_q,
           W_msg_pb, W_msg_bq, W_msg_pq, ln_ball_g, ln_ball_b, ln_q_g, ln_q_b,
           P1_w, P1_b, ln_p1_g, ln_p1_b, P2_w, P2_b, ln_p2_g, ln_p2_b,
           P3_w, P3_b):
    player_ids = player_ids.astype(i32)
    src_pb = src_pb.astype(i32)
    dst_pb = dst_pb.astype(i32)
    src_bq = src_bq.astype(i32)
    dst_bq = dst_bq.astype(i32)
    src_pq = src_pq.astype(i32)
    dst_pq = dst_pq.astype(i32)

    pt_pad = jnp.pad(player_table, ((0, NTAB - 5000), (0, 0)))
    enc0, enc1 = _k1(pt_pad, W_p, b_p.reshape(1, H))

    npad = E_PB_PAD - E_PB
    npad_q = E_PQ_PAD - E_PQ
    pid_p = jnp.concatenate([player_ids, jnp.full((NPID - 50000,), 0, i32)])
    srcpb_p = jnp.concatenate([src_pb, jnp.full((npad,), 0, i32)])
    dstpb_p = jnp.concatenate([dst_pb, jnp.full((npad,), NB, i32)])
    srcpq_p = jnp.concatenate([src_pq, jnp.full((npad_q,), 0, i32)])
    dstpq_p = jnp.concatenate([dst_pq, jnp.full((npad_q,), NQ, i32)])

    def chunked(src, dst):
        return jnp.stack([src.reshape(-1, C), dst.reshape(-1, C)], axis=1)

    pbe = chunked(srcpb_p, dstpb_p)
    bqe = chunked(src_bq, dst_bq)
    pqe = chunked(srcpq_p, dstpq_p)

    zeros50 = jnp.zeros((ZROWS, 32), f32)
    ones256 = jnp.ones((C, 32), f32)
    zerosq = jnp.zeros((QROWS, H), f32)

    encf0, encf1 = _k1b(enc0, enc1, pid_p)
    spb, degb, sq, degq2, degq1 = _k2(
        encf0, encf1, pbe, bqe, pqe, zeros50, ones256)

    hb1, hb2, hb3 = _k3(ball_feat, spb, degb, W_b, b_b.reshape(1, H),
                        W_msg_pb, ln_ball_g, ln_ball_b)

    T = _k4(hb1, hb2, hb3, bqe, zerosq)

    P3_w8 = jnp.pad(P3_w, ((0, 0), (0, 1)))
    P3_b8 = jnp.pad(P3_b, ((0, 1))).reshape(1, 8)
    out8 = _k5(query_feat, T, sq, degq1, degq2,
               W_q, b_q.reshape(1, H), W_msg_bq, W_msg_pq, ln_q_g, ln_q_b,
               P1_w, P1_b.reshape(1, H), ln_p1_g.reshape(1, H),
               ln_p1_b.reshape(1, H), P2_w, P2_b.reshape(1, 32),
               ln_p2_g.reshape(1, 32), ln_p2_b.reshape(1, 32), P3_w8, P3_b8)
    return out8[:, :7]
